# Initial kernel scaffold; baseline (speedup 1.0000x reference)
import jax
import jax.numpy as jnp
from jax.experimental import pallas as pl


def kernel(x, Wr, Wg, Wu, Wd):
    raise NotImplementedError("write your pallas kernel here")


# speedup vs baseline: 35.8703x; 35.8703x over previous
import jax
import jax.numpy as jnp
from jax.experimental import pallas as pl
from jax.experimental.pallas import tpu as pltpu

_D_MODEL = 1024
_D_FFN = 2048
_N_EXP = 8
_NELT = _D_FFN * _D_MODEL
_K1 = _NELT // 2 - 1


def _median_body(w_ref, a_ref):
    v = jax.lax.bitcast_convert_type(jnp.abs(w_ref[0]), jnp.int32)

    def step(i, p):
        t = p | jnp.left_shift(jnp.int32(1), jnp.int32(30) - i)
        c = jnp.sum((v < t).astype(jnp.int32))
        return jnp.where(c <= _K1, t, p)

    s_a = jax.lax.fori_loop(0, 31, step, jnp.int32(0))
    c_le = jnp.sum((v <= s_a).astype(jnp.int32))
    bigger = jnp.where(v > s_a, v, jnp.int32(0x7FFFFFFF))
    s_b = jnp.where(c_le >= _K1 + 2, s_a, jnp.min(bigger))
    fa = jax.lax.bitcast_convert_type(s_a, jnp.float32)
    fb = jax.lax.bitcast_convert_type(s_b, jnp.float32)
    a_ref[pl.program_id(0)] = (fa + fb) * 0.5


def _alphas(w):
    return pl.pallas_call(
        _median_body,
        grid=(_N_EXP,),
        in_specs=[pl.BlockSpec((1, _D_FFN, _D_MODEL), lambda e: (e, 0, 0))],
        out_specs=pl.BlockSpec((_N_EXP,), lambda e: (0,),
                               memory_space=pltpu.SMEM),
        out_shape=jax.ShapeDtypeStruct((_N_EXP,), jnp.float32),
    )(w)


def _quant(w, a):
    return jnp.where(w > a, 1.0, jnp.where(w < -a, -1.0, 0.0))


def _ffn_body(alpha_ref, x_ref, wr_ref, wg_ref, wu_ref, wd_ref, out_ref,
              coef_ref):
    e = pl.program_id(0)
    f = pl.program_id(1)
    xv = x_ref[...]
    s = xv.shape[0]

    @pl.when((e == 0) & (f == 0))
    def _router():
        wr = wr_ref[...]
        logits = jax.lax.dot_general(
            xv, wr, (((1,), (1,)), ((), ())),
            preferred_element_type=jnp.float32)
        ids = jax.lax.broadcasted_iota(jnp.int32, logits.shape, 1)
        m1 = jnp.max(logits, axis=1, keepdims=True)
        i1 = jnp.min(jnp.where(logits == m1, ids, _N_EXP), axis=1,
                     keepdims=True)
        rest = jnp.where(ids == i1, -jnp.inf, logits)
        m2 = jnp.max(rest, axis=1, keepdims=True)
        i2 = jnp.min(jnp.where(rest == m2, ids, _N_EXP), axis=1,
                     keepdims=True)
        r = jnp.exp(m2 - m1)
        s1 = 1.0 / (1.0 + r)
        s2 = r / (1.0 + r)
        coef_ref[...] = (jnp.where(ids == i1, s1, 0.0)
                         + jnp.where(ids == i2, s2, 0.0))

    ag = alpha_ref[0, e]
    au = alpha_ref[1, e]
    ad = alpha_ref[2, e]
    qg = _quant(wg_ref[0], ag)
    qu = _quant(wu_ref[0], au)
    qd = _quant(wd_ref[0], ad)
    g = jax.lax.dot_general(xv, qg, (((1,), (1,)), ((), ())),
                            preferred_element_type=jnp.float32)
    u = jax.lax.dot_general(xv, qu, (((1,), (1,)), ((), ())),
                            preferred_element_type=jnp.float32)
    h = g * jax.nn.sigmoid(g) * u
    o = jax.lax.dot_general(h, qd, (((1,), (1,)), ((), ())),
                            preferred_element_type=jnp.float32)
    ids8 = jax.lax.broadcasted_iota(jnp.int32, (s, _N_EXP), 1)
    ce = jnp.sum(jnp.where(ids8 == e, coef_ref[...], 0.0), axis=1,
                 keepdims=True)

    @pl.when((e == 0) & (f == 0))
    def _init():
        out_ref[...] = jnp.zeros_like(out_ref)

    out_ref[...] += o * ce


def _moe_ffn(xf, Wr, Wg, Wu, Wd, alphas):
    s = xf.shape[0]
    fsplit = 2
    fb = _D_FFN // fsplit
    return pl.pallas_call(
        _ffn_body,
        grid=(_N_EXP, fsplit),
        in_specs=[
            pl.BlockSpec(memory_space=pltpu.SMEM),
            pl.BlockSpec((s, _D_MODEL), lambda e, f: (0, 0)),
            pl.BlockSpec((_N_EXP, _D_MODEL), lambda e, f: (0, 0)),
            pl.BlockSpec((1, fb, _D_MODEL), lambda e, f: (e, f, 0)),
            pl.BlockSpec((1, fb, _D_MODEL), lambda e, f: (e, f, 0)),
            pl.BlockSpec((1, _D_MODEL, fb), lambda e, f: (e, 0, f)),
        ],
        out_specs=pl.BlockSpec((s, _D_MODEL), lambda e, f: (0, 0)),
        out_shape=jax.ShapeDtypeStruct((s, _D_MODEL), jnp.float32),
        scratch_shapes=[pltpu.VMEM((s, _N_EXP), jnp.float32)],
    )(alphas, xf, Wr, Wg, Wu, Wd)


def kernel(x, Wr, Wg, Wu, Wd):
    B, T, D = x.shape
    xf = x.reshape(-1, D)
    ag = _alphas(Wg)
    au = _alphas(Wu)
    ad = _alphas(Wd.reshape(_N_EXP, _D_FFN, _D_MODEL))
    alphas = jnp.stack([ag, au, ad])
    out = _moe_ffn(xf, Wr, Wg, Wu, Wd, alphas)
    return out.reshape(B, T, D)


# speedup vs baseline: 72.2601x; 2.0145x over previous
import jax
import jax.numpy as jnp
from jax.experimental import pallas as pl
from jax.experimental.pallas import tpu as pltpu

_D_MODEL = 1024
_D_FFN = 2048
_N_EXP = 8
_NELT = _D_FFN * _D_MODEL
_K1 = _NELT // 2 - 1


def _f2i(x):
    return jax.lax.bitcast_convert_type(x, jnp.int32)


def _i2f(x):
    return jax.lax.bitcast_convert_type(x, jnp.float32)


def _median_body(w_ref, a_ref, ab_ref):
    ab_ref[...] = jnp.abs(w_ref[0])
    a = ab_ref[...]

    def count_lt(tf):
        return jnp.sum((a < tf).astype(jnp.int32))

    mn = jnp.min(a)
    mx = jnp.max(a)
    lo = _f2i(mn)
    hi = _f2i(mx) + 1
    k = jnp.int32(_K1)

    def cond(carry):
        lo_, hi_, cl, ch, _ = carry
        return (hi_ - lo_ > 1) & (cl != k) & (ch != k + 1)

    def body(carry):
        lo_, hi_, cl, ch, it = carry
        fl = _i2f(lo_)
        fh = _i2f(hi_)
        frac = (k.astype(jnp.float32) + 0.5 - cl.astype(jnp.float32)) / (
            ch.astype(jnp.float32) - cl.astype(jnp.float32))
        t_interp = _f2i(fl + (fh - fl) * frac)
        t_bisect = lo_ + (hi_ - lo_) // 2
        t = jnp.where((it < 12) | (it % 2 == 0), t_interp, t_bisect)
        t = jnp.clip(t, lo_ + 1, hi_ - 1)
        c = count_lt(_i2f(t))
        take = c <= k
        return (jnp.where(take, t, lo_), jnp.where(take, hi_, t),
                jnp.where(take, c, cl), jnp.where(take, ch, c), it + 1)

    lo, hi, cl, ch, _ = jax.lax.while_loop(
        cond, body, (lo, hi, jnp.int32(0), jnp.int32(_NELT), jnp.int32(0)))

    s_a = jax.lax.cond(
        cl == k,
        lambda: _f2i(jnp.min(jnp.where(a >= _i2f(lo), a, jnp.inf))),
        lambda: jax.lax.cond(
            ch == k + 1,
            lambda: _f2i(jnp.max(jnp.where(a < _i2f(hi), a, -jnp.inf))),
            lambda: lo))

    fa = _i2f(s_a)
    c_le = jnp.sum((a <= fa).astype(jnp.int32))
    fb = jax.lax.cond(
        c_le >= k + 2,
        lambda: fa,
        lambda: jnp.min(jnp.where(a > fa, a, jnp.inf)))
    a_ref[pl.program_id(0)] = (fa + fb) * 0.5


def _alphas(w):
    return pl.pallas_call(
        _median_body,
        grid=(_N_EXP,),
        in_specs=[pl.BlockSpec((1, _D_FFN, _D_MODEL), lambda e: (e, 0, 0))],
        out_specs=pl.BlockSpec((_N_EXP,), lambda e: (0,),
                               memory_space=pltpu.SMEM),
        out_shape=jax.ShapeDtypeStruct((_N_EXP,), jnp.float32),
        scratch_shapes=[pltpu.VMEM((_D_FFN, _D_MODEL), jnp.float32)],
    )(w)


def _quant(w, a):
    return jnp.where(w > a, 1.0, jnp.where(w < -a, -1.0, 0.0))


def _ffn_body(alpha_ref, x_ref, wr_ref, wg_ref, wu_ref, wd_ref, out_ref,
              coef_ref):
    e = pl.program_id(0)
    f = pl.program_id(1)
    xv = x_ref[...]
    s = xv.shape[0]

    @pl.when((e == 0) & (f == 0))
    def _router():
        wr = wr_ref[...]
        logits = jax.lax.dot_general(
            xv, wr, (((1,), (1,)), ((), ())),
            preferred_element_type=jnp.float32)
        ids = jax.lax.broadcasted_iota(jnp.int32, logits.shape, 1)
        m1 = jnp.max(logits, axis=1, keepdims=True)
        i1 = jnp.min(jnp.where(logits == m1, ids, _N_EXP), axis=1,
                     keepdims=True)
        rest = jnp.where(ids == i1, -jnp.inf, logits)
        m2 = jnp.max(rest, axis=1, keepdims=True)
        i2 = jnp.min(jnp.where(rest == m2, ids, _N_EXP), axis=1,
                     keepdims=True)
        r = jnp.exp(m2 - m1)
        s1 = 1.0 / (1.0 + r)
        s2 = r / (1.0 + r)
        coef_ref[...] = (jnp.where(ids == i1, s1, 0.0)
                         + jnp.where(ids == i2, s2, 0.0))

    ag = alpha_ref[0, e]
    au = alpha_ref[1, e]
    ad = alpha_ref[2, e]
    qg = _quant(wg_ref[0], ag)
    qu = _quant(wu_ref[0], au)
    qd = _quant(wd_ref[0], ad)
    g = jax.lax.dot_general(xv, qg, (((1,), (1,)), ((), ())),
                            preferred_element_type=jnp.float32)
    u = jax.lax.dot_general(xv, qu, (((1,), (1,)), ((), ())),
                            preferred_element_type=jnp.float32)
    h = g * jax.nn.sigmoid(g) * u
    o = jax.lax.dot_general(h, qd, (((1,), (1,)), ((), ())),
                            preferred_element_type=jnp.float32)
    ids8 = jax.lax.broadcasted_iota(jnp.int32, (s, _N_EXP), 1)
    ce = jnp.sum(jnp.where(ids8 == e, coef_ref[...], 0.0), axis=1,
                 keepdims=True)

    @pl.when((e == 0) & (f == 0))
    def _init():
        out_ref[...] = jnp.zeros_like(out_ref)

    out_ref[...] += o * ce


def _moe_ffn(xf, Wr, Wg, Wu, Wd, alphas):
    s = xf.shape[0]
    fsplit = 2
    fb = _D_FFN // fsplit
    return pl.pallas_call(
        _ffn_body,
        grid=(_N_EXP, fsplit),
        in_specs=[
            pl.BlockSpec(memory_space=pltpu.SMEM),
            pl.BlockSpec((s, _D_MODEL), lambda e, f: (0, 0)),
            pl.BlockSpec((_N_EXP, _D_MODEL), lambda e, f: (0, 0)),
            pl.BlockSpec((1, fb, _D_MODEL), lambda e, f: (e, f, 0)),
            pl.BlockSpec((1, fb, _D_MODEL), lambda e, f: (e, f, 0)),
            pl.BlockSpec((1, _D_MODEL, fb), lambda e, f: (e, 0, f)),
        ],
        out_specs=pl.BlockSpec((s, _D_MODEL), lambda e, f: (0, 0)),
        out_shape=jax.ShapeDtypeStruct((s, _D_MODEL), jnp.float32),
        scratch_shapes=[pltpu.VMEM((s, _N_EXP), jnp.float32)],
    )(alphas, xf, Wr, Wg, Wu, Wd)


def kernel(x, Wr, Wg, Wu, Wd):
    B, T, D = x.shape
    xf = x.reshape(-1, D)
    ag = _alphas(Wg)
    au = _alphas(Wu)
    ad = _alphas(Wd.reshape(_N_EXP, _D_FFN, _D_MODEL))
    alphas = jnp.stack([ag, au, ad])
    out = _moe_ffn(xf, Wr, Wg, Wu, Wd, alphas)
    return out.reshape(B, T, D)


# speedup vs baseline: 122.6514x; 1.6974x over previous
import jax
import jax.numpy as jnp
from jax.experimental import pallas as pl
from jax.experimental.pallas import tpu as pltpu

_D_MODEL = 1024
_D_FFN = 2048
_N_EXP = 8
_NELT = _D_FFN * _D_MODEL
_K1 = _NELT // 2 - 1


def _f2i(x):
    return jax.lax.bitcast_convert_type(x, jnp.int32)


def _i2f(x):
    return jax.lax.bitcast_convert_type(x, jnp.float32)


def _median_body(w_ref, a_ref, ab_ref):
    ab_ref[...] = jnp.abs(w_ref[0])
    nchain = 8
    rows = _D_FFN // nchain

    def parts():
        return [ab_ref[pl.ds(j * rows, rows), :] for j in range(nchain)]

    def _tree(vals, op):
        while len(vals) > 1:
            vals = [op(vals[i], vals[i + 1]) if i + 1 < len(vals) else vals[i]
                    for i in range(0, len(vals), 2)]
        return vals[0]

    def count_lt(tf):
        return _tree([jnp.sum((p < tf).astype(jnp.int32)) for p in parts()],
                     jnp.add)

    def count_le(tf):
        return _tree([jnp.sum((p <= tf).astype(jnp.int32)) for p in parts()],
                     jnp.add)

    def masked_min(tf, strict):
        def one(p):
            m = (p > tf) if strict else (p >= tf)
            return jnp.min(jnp.where(m, p, jnp.inf))
        return _tree([one(p) for p in parts()], jnp.minimum)

    def masked_max_lt(tf):
        return _tree([jnp.max(jnp.where(p < tf, p, -jnp.inf))
                      for p in parts()], jnp.maximum)

    mn = _tree([jnp.min(p) for p in parts()], jnp.minimum)
    mx = _tree([jnp.max(p) for p in parts()], jnp.maximum)
    lo = _f2i(mn)
    hi = _f2i(mx) + 1
    k = jnp.int32(_K1)

    def cond(carry):
        lo_, hi_, cl, ch, _ = carry
        return (hi_ - lo_ > 1) & (cl != k) & (ch != k + 1)

    def body(carry):
        lo_, hi_, cl, ch, it = carry
        fl = _i2f(lo_)
        fh = _i2f(hi_)
        frac = (k.astype(jnp.float32) + 0.5 - cl.astype(jnp.float32)) / (
            ch.astype(jnp.float32) - cl.astype(jnp.float32))
        t_interp = _f2i(fl + (fh - fl) * frac)
        t_bisect = lo_ + (hi_ - lo_) // 2
        t = jnp.where((it < 12) | (it % 2 == 0), t_interp, t_bisect)
        t = jnp.clip(t, lo_ + 1, hi_ - 1)
        c = count_lt(_i2f(t))
        take = c <= k
        return (jnp.where(take, t, lo_), jnp.where(take, hi_, t),
                jnp.where(take, c, cl), jnp.where(take, ch, c), it + 1)

    lo, hi, cl, ch, _ = jax.lax.while_loop(
        cond, body, (lo, hi, jnp.int32(0), jnp.int32(_NELT), jnp.int32(0)))

    s_a = jax.lax.cond(
        cl == k,
        lambda: _f2i(masked_min(_i2f(lo), strict=False)),
        lambda: jax.lax.cond(
            ch == k + 1,
            lambda: _f2i(masked_max_lt(_i2f(hi))),
            lambda: lo))

    fa = _i2f(s_a)
    c_le = count_le(fa)
    fb = jax.lax.cond(
        c_le >= k + 2,
        lambda: fa,
        lambda: masked_min(fa, strict=True))
    a_ref[pl.program_id(0)] = (fa + fb) * 0.5


def _alphas(w):
    return pl.pallas_call(
        _median_body,
        grid=(_N_EXP,),
        in_specs=[pl.BlockSpec((1, _D_FFN, _D_MODEL), lambda e: (e, 0, 0))],
        out_specs=pl.BlockSpec((_N_EXP,), lambda e: (0,),
                               memory_space=pltpu.SMEM),
        out_shape=jax.ShapeDtypeStruct((_N_EXP,), jnp.float32),
        scratch_shapes=[pltpu.VMEM((_D_FFN, _D_MODEL), jnp.float32)],
    )(w)


def _quant(w, a):
    return jnp.where(w > a, 1.0, jnp.where(w < -a, -1.0, 0.0))


def _ffn_body(alpha_ref, x_ref, wr_ref, wg_ref, wu_ref, wd_ref, out_ref,
              coef_ref):
    e = pl.program_id(0)
    f = pl.program_id(1)
    xv = x_ref[...]
    s = xv.shape[0]

    @pl.when((e == 0) & (f == 0))
    def _router():
        wr = wr_ref[...]
        logits = jax.lax.dot_general(
            xv, wr, (((1,), (1,)), ((), ())),
            preferred_element_type=jnp.float32)
        ids = jax.lax.broadcasted_iota(jnp.int32, logits.shape, 1)
        m1 = jnp.max(logits, axis=1, keepdims=True)
        i1 = jnp.min(jnp.where(logits == m1, ids, _N_EXP), axis=1,
                     keepdims=True)
        rest = jnp.where(ids == i1, -jnp.inf, logits)
        m2 = jnp.max(rest, axis=1, keepdims=True)
        i2 = jnp.min(jnp.where(rest == m2, ids, _N_EXP), axis=1,
                     keepdims=True)
        r = jnp.exp(m2 - m1)
        s1 = 1.0 / (1.0 + r)
        s2 = r / (1.0 + r)
        coef_ref[...] = (jnp.where(ids == i1, s1, 0.0)
                         + jnp.where(ids == i2, s2, 0.0))

    ag = alpha_ref[0, e]
    au = alpha_ref[1, e]
    ad = alpha_ref[2, e]
    qg = _quant(wg_ref[0], ag)
    qu = _quant(wu_ref[0], au)
    qd = _quant(wd_ref[0], ad)
    g = jax.lax.dot_general(xv, qg, (((1,), (1,)), ((), ())),
                            preferred_element_type=jnp.float32)
    u = jax.lax.dot_general(xv, qu, (((1,), (1,)), ((), ())),
                            preferred_element_type=jnp.float32)
    h = g * jax.nn.sigmoid(g) * u
    o = jax.lax.dot_general(h, qd, (((1,), (1,)), ((), ())),
                            preferred_element_type=jnp.float32)
    ids8 = jax.lax.broadcasted_iota(jnp.int32, (s, _N_EXP), 1)
    ce = jnp.sum(jnp.where(ids8 == e, coef_ref[...], 0.0), axis=1,
                 keepdims=True)

    @pl.when((e == 0) & (f == 0))
    def _init():
        out_ref[...] = jnp.zeros_like(out_ref)

    out_ref[...] += o * ce


def _moe_ffn(xf, Wr, Wg, Wu, Wd, alphas):
    s = xf.shape[0]
    fsplit = 2
    fb = _D_FFN // fsplit
    return pl.pallas_call(
        _ffn_body,
        grid=(_N_EXP, fsplit),
        in_specs=[
            pl.BlockSpec(memory_space=pltpu.SMEM),
            pl.BlockSpec((s, _D_MODEL), lambda e, f: (0, 0)),
            pl.BlockSpec((_N_EXP, _D_MODEL), lambda e, f: (0, 0)),
            pl.BlockSpec((1, fb, _D_MODEL), lambda e, f: (e, f, 0)),
            pl.BlockSpec((1, fb, _D_MODEL), lambda e, f: (e, f, 0)),
            pl.BlockSpec((1, _D_MODEL, fb), lambda e, f: (e, 0, f)),
        ],
        out_specs=pl.BlockSpec((s, _D_MODEL), lambda e, f: (0, 0)),
        out_shape=jax.ShapeDtypeStruct((s, _D_MODEL), jnp.float32),
        scratch_shapes=[pltpu.VMEM((s, _N_EXP), jnp.float32)],
    )(alphas, xf, Wr, Wg, Wu, Wd)


def kernel(x, Wr, Wg, Wu, Wd):
    B, T, D = x.shape
    xf = x.reshape(-1, D)
    ag = _alphas(Wg)
    au = _alphas(Wu)
    ad = _alphas(Wd.reshape(_N_EXP, _D_FFN, _D_MODEL))
    alphas = jnp.stack([ag, au, ad])
    out = _moe_ffn(xf, Wr, Wg, Wu, Wd, alphas)
    return out.reshape(B, T, D)


# speedup vs baseline: 171.3019x; 1.3967x over previous
import functools

import jax
from jax import lax
import jax.numpy as jnp
from jax.experimental import pallas as pl
from jax.experimental.pallas import tpu as pltpu
from jax.experimental.pallas import tpu_sc as plsc

_D_MODEL = 1024
_D_FFN = 2048
_N_EXP = 8
_NELT = _D_FFN * _D_MODEL
_K1 = _NELT // 2 - 1


def _f2i(x):
    return jax.lax.bitcast_convert_type(x, jnp.int32)


def _i2f(x):
    return jax.lax.bitcast_convert_type(x, jnp.float32)


def _median_body(warm_ref, w_ref, a_ref, ab_ref, prev_ref):
    e = pl.program_id(0)
    ab_ref[...] = jnp.abs(w_ref[0])
    nchain = 8
    rows = _D_FFN // nchain
    k = jnp.int32(_K1)

    def parts():
        return [ab_ref[pl.ds(j * rows, rows), :] for j in range(nchain)]

    def _tree(vals, op):
        while len(vals) > 1:
            vals = [op(vals[i], vals[i + 1]) if i + 1 < len(vals) else vals[i]
                    for i in range(0, len(vals), 2)]
        return vals[0]

    def _treemap(fns, ps):
        outs = []
        for fn, comb in fns:
            outs.append(_tree([fn(p) for p in ps], comb))
        return outs

    def count_lt(tf):
        return _tree([jnp.sum((p < tf).astype(jnp.int32)) for p in parts()],
                     jnp.add)

    warm = jnp.where(e == 0, warm_ref[0], prev_ref[0])
    mn, mx, c0 = _treemap(
        [(jnp.min, jnp.minimum), (jnp.max, jnp.maximum),
         (lambda p: jnp.sum((p < warm).astype(jnp.int32)), jnp.add)],
        parts())
    t0 = _f2i(warm)
    take0 = c0 <= k
    lo = jnp.where(take0, t0, _f2i(mn))
    cl = jnp.where(take0, c0, jnp.int32(0))
    hi = jnp.where(take0, _f2i(mx) + 1, t0)
    ch = jnp.where(take0, jnp.int32(_NELT), c0)
    sig = mx * jnp.float32(1.0 / 5.2)

    def cond(carry):
        lo_, hi_, cl_, ch_, _, _, _ = carry
        return (hi_ - lo_ > 1) & (cl_ != k) & (ch_ != k + 1)

    def body(carry):
        lo_, hi_, cl_, ch_, tp, cp, it = carry
        t_newton = _f2i(_i2f(tp) + (k.astype(jnp.float32) + 0.5
                                    - cp.astype(jnp.float32))
                        * sig * jnp.float32(1.0 / (0.635 * _NELT)))
        fl = _i2f(lo_)
        fh = _i2f(hi_)
        frac = (k.astype(jnp.float32) + 0.5 - cl_.astype(jnp.float32)) / (
            ch_.astype(jnp.float32) - cl_.astype(jnp.float32))
        t_interp = _f2i(fl + (fh - fl) * frac)
        t_bisect = lo_ + (hi_ - lo_) // 2
        t = jnp.where(it == 1, t_newton,
                      jnp.where((it < 12) | (it % 2 == 0),
                                t_interp, t_bisect))
        t = jnp.clip(t, lo_ + 1, hi_ - 1)
        c = count_lt(_i2f(t))
        take = c <= k
        return (jnp.where(take, t, lo_), jnp.where(take, hi_, t),
                jnp.where(take, c, cl_), jnp.where(take, ch_, c),
                t, c, it + 1)

    lo, hi, cl, ch, _, _, _ = jax.lax.while_loop(
        cond, body, (lo, hi, cl, ch, t0, c0, jnp.int32(1)))

    def eqcnt_min2(fa):
        return _treemap(
            [(lambda p: jnp.sum((p == fa).astype(jnp.int32)), jnp.add),
             (lambda p: jnp.min(jnp.where(p > fa, p, jnp.inf)), jnp.minimum)],
            parts())

    def case_a():
        m1 = _tree([jnp.min(jnp.where(p >= _i2f(lo), p, jnp.inf))
                    for p in parts()], jnp.minimum)
        cnt_eq, m2 = eqcnt_min2(m1)
        return m1, jnp.where(cnt_eq >= 2, m1, m2)

    def case_b():
        fh = _i2f(hi)
        m_lt, m_ge = _treemap(
            [(lambda p: jnp.max(jnp.where(p < fh, p, -jnp.inf)), jnp.maximum),
             (lambda p: jnp.min(jnp.where(p >= fh, p, jnp.inf)), jnp.minimum)],
            parts())
        return m_lt, m_ge

    def case_c():
        fa = _i2f(lo)
        cnt_eq, m2 = eqcnt_min2(fa)
        c_le = cl + cnt_eq
        return fa, jnp.where(c_le >= k + 2, fa, m2)

    fa, fb = jax.lax.cond(
        cl == k, case_a, lambda: jax.lax.cond(ch == k + 1, case_b, case_c))
    prev_ref[0] = fa
    a_ref[e] = (fa + fb) * 0.5


def _alphas(w, warm):
    return pl.pallas_call(
        _median_body,
        grid=(_N_EXP,),
        in_specs=[
            pl.BlockSpec(memory_space=pltpu.SMEM),
            pl.BlockSpec((1, _D_FFN, _D_MODEL), lambda e: (e, 0, 0)),
        ],
        out_specs=pl.BlockSpec((_N_EXP,), lambda e: (0,),
                               memory_space=pltpu.SMEM),
        out_shape=jax.ShapeDtypeStruct((_N_EXP,), jnp.float32),
        scratch_shapes=[pltpu.VMEM((_D_FFN, _D_MODEL), jnp.float32),
                        pltpu.SMEM((1,), jnp.float32)],
    )(jnp.reshape(warm, (1,)).astype(jnp.float32), w)


def _quant(w, a):
    return jnp.where(w > a, 1.0, jnp.where(w < -a, -1.0, 0.0))


def _logits_body(x_ref, wr_ref, out_ref):
    out_ref[...] = jax.lax.dot_general(
        wr_ref[...], x_ref[...], (((1,), (1,)), ((), ())),
        preferred_element_type=jnp.float32)


def _logitsT(xf, Wr):
    s = xf.shape[0]
    return pl.pallas_call(
        _logits_body,
        out_shape=jax.ShapeDtypeStruct((_N_EXP, s), jnp.float32),
    )(xf, Wr)


_SC_NC = 2
_SC_NS = 16
_SC_L = 16


def _router_sc(logitsT):
    s = logitsT.shape[1]
    per = s // (_SC_NC * _SC_NS)
    mesh = plsc.VectorSubcoreMesh(core_axis_name="c", subcore_axis_name="s")

    @functools.partial(
        pl.kernel, mesh=mesh,
        out_type=jax.ShapeDtypeStruct((_N_EXP, s), jnp.float32),
        scratch_types=[pltpu.VMEM((_N_EXP, per), jnp.float32),
                       pltpu.VMEM((_N_EXP, per), jnp.float32)],
    )
    def body(l_hbm, o_hbm, lv, cv):
        wid = lax.axis_index("s") * _SC_NC + lax.axis_index("c")
        base = wid * per
        for e in range(_N_EXP):
            pltpu.sync_copy(l_hbm.at[e, pl.ds(base, per)], lv.at[e])
        v = [lv[e] for e in range(_N_EXP)]
        m1 = v[0]
        for e in range(1, _N_EXP):
            m1 = jnp.maximum(m1, v[e])
        i1 = jnp.full((_SC_L,), _N_EXP - 1, jnp.int32)
        for e in range(_N_EXP - 2, -1, -1):
            i1 = jnp.where(v[e] == m1, jnp.int32(e), i1)
        neg = jnp.full((_SC_L,), -jnp.inf, jnp.float32)
        rest = [jnp.where(i1 == e, neg, v[e]) for e in range(_N_EXP)]
        m2 = rest[0]
        for e in range(1, _N_EXP):
            m2 = jnp.maximum(m2, rest[e])
        i2 = jnp.full((_SC_L,), _N_EXP - 1, jnp.int32)
        for e in range(_N_EXP - 2, -1, -1):
            i2 = jnp.where(rest[e] == m2, jnp.int32(e), i2)
        r = jnp.exp(m2 - m1)
        s1 = 1.0 / (1.0 + r)
        s2 = r * s1
        zero = jnp.zeros((_SC_L,), jnp.float32)
        for e in range(_N_EXP):
            cv[e] = jnp.where(i1 == e, s1, jnp.where(i2 == e, s2, zero))
        for e in range(_N_EXP):
            pltpu.sync_copy(cv.at[e], o_hbm.at[e, pl.ds(base, per)])

    return body(logitsT)


def _ffn_body(alpha_ref, x_ref, coef_ref, wg_ref, wu_ref, wd_ref, out_ref):
    e = pl.program_id(0)
    f = pl.program_id(1)
    xv = x_ref[...]
    s = xv.shape[0]

    ag = alpha_ref[0, e]
    au = alpha_ref[1, e]
    ad = alpha_ref[2, e]
    qg = _quant(wg_ref[0], ag)
    qu = _quant(wu_ref[0], au)
    qd = _quant(wd_ref[0], ad)
    g = jax.lax.dot_general(xv, qg, (((1,), (1,)), ((), ())),
                            preferred_element_type=jnp.float32)
    u = jax.lax.dot_general(xv, qu, (((1,), (1,)), ((), ())),
                            preferred_element_type=jnp.float32)
    h = g * jax.nn.sigmoid(g) * u
    o = jax.lax.dot_general(h, qd, (((1,), (1,)), ((), ())),
                            preferred_element_type=jnp.float32)
    ids8 = jax.lax.broadcasted_iota(jnp.int32, (s, _N_EXP), 1)
    ce = jnp.sum(jnp.where(ids8 == e, coef_ref[...], 0.0), axis=1,
                 keepdims=True)

    @pl.when((e == 0) & (f == 0))
    def _init():
        out_ref[...] = jnp.zeros_like(out_ref)

    out_ref[...] += o * ce


def _moe_ffn(xf, coef, Wg, Wu, Wd, alphas):
    s = xf.shape[0]
    fsplit = 2
    fb = _D_FFN // fsplit
    return pl.pallas_call(
        _ffn_body,
        grid=(_N_EXP, fsplit),
        in_specs=[
            pl.BlockSpec(memory_space=pltpu.SMEM),
            pl.BlockSpec((s, _D_MODEL), lambda e, f: (0, 0)),
            pl.BlockSpec((s, _N_EXP), lambda e, f: (0, 0)),
            pl.BlockSpec((1, fb, _D_MODEL), lambda e, f: (e, f, 0)),
            pl.BlockSpec((1, fb, _D_MODEL), lambda e, f: (e, f, 0)),
            pl.BlockSpec((1, _D_MODEL, fb), lambda e, f: (e, 0, f)),
        ],
        out_specs=pl.BlockSpec((s, _D_MODEL), lambda e, f: (0, 0)),
        out_shape=jax.ShapeDtypeStruct((s, _D_MODEL), jnp.float32),
    )(alphas, xf, coef, Wg, Wu, Wd)


def kernel(x, Wr, Wg, Wu, Wd):
    B, T, D = x.shape
    xf = x.reshape(-1, D)
    coefT = _router_sc(_logitsT(xf, Wr))
    ag = _alphas(Wg, jnp.float32(0.6745 * 1.5 / 32.0))
    au = _alphas(Wu, ag[-1])
    ad = _alphas(Wd.reshape(_N_EXP, _D_FFN, _D_MODEL),
                 au[-1] * jnp.float32(0.70710678))
    alphas = jnp.stack([ag, au, ad])
    out = _moe_ffn(xf, coefT.T, Wg, Wu, Wd, alphas)
    return out.reshape(B, T, D)
